# baseline (speedup 1.0000x reference)
import jax
import jax.numpy as jnp
import numpy as np
from jax import lax
from jax.experimental import pallas as pl
from jax.experimental.pallas import tpu as pltpu
from jax.experimental.pallas import tpu_sc as plsc

N_FIELDS = 26
VOCAB = 100000
EMB_DIM = 64
BATCH = 4096
N_CONT = 13
BN_EPS = 1e-5

NC = 2
NS = 16
NW = NC * NS
T_ROWS = N_FIELDS * EMB_DIM
ROWS_PER_W = T_ROWS // NW
OUT_W = T_ROWS + N_CONT
INV_STD = float(1.0 / np.sqrt(1.0 + BN_EPS))
N_GRP = BATCH // 16
HALF = 50048
BMAIN = 49920
VTAIL = VOCAB - HALF - BMAIN


def _sc_body(t2_ref, tailp_ref, catt_ref, contt_ref, gam_ref, bet_ref,
             outt_ref, buf_a, buf_b, vidx, crow, orow_a, orow_b, gb_v,
             sem_a, sem_b, osem):
  wid = lax.axis_index("s") * NC + lax.axis_index("c")
  base = wid * ROWS_PER_W
  iota = lax.iota(jnp.int32, 16)
  zeros = jnp.zeros((16,), jnp.int32)

  def dma_a(r):
    return pltpu.async_copy(
        t2_ref.at[pl.ds(r, 1), pl.ds(0, HALF)], buf_a, sem_a)

  def dma_b(r):
    return [
        pltpu.async_copy(t2_ref.at[pl.ds(r, 1), pl.ds(HALF, BMAIN)],
                         buf_b.at[:, pl.ds(0, BMAIN)], sem_b),
        pltpu.async_copy(tailp_ref.at[pl.ds(r, 1), :],
                         buf_b.at[:, pl.ds(BMAIN, 128)], sem_b),
    ]

  cp_a = dma_a(base)
  cp_bs = dma_b(base)
  pltpu.sync_copy(catt_ref.at[pl.ds(base // EMB_DIM, 1), :], vidx)

  out_cps = [None, None]
  for c in range(ROWS_PER_W):
    r = base + c
    slot = c % 2
    ob = orow_a if slot == 0 else orow_b
    if out_cps[slot] is not None:
      out_cps[slot].wait()
    cp_a.wait()

    def extract_a(s, _, ob=ob):
      for u in range(4):
        g = s * 4 + u
        pos = iota + g * 16
        v16 = vidx[0, pl.ds(g * 16, 16)]
        vals = plsc.load_gather(buf_a, [zeros, jnp.minimum(v16, HALF - 1)])
        plsc.store_scatter(ob, [zeros, pos], vals, mask=v16 < HALF)
      return 0

    lax.fori_loop(0, N_GRP // 4, extract_a, 0)
    if c + 1 < ROWS_PER_W:
      cp_a = dma_a(r + 1)
    for cp in cp_bs:
      cp.wait()

    def extract_b(s, _, ob=ob):
      for u in range(4):
        g = s * 4 + u
        pos = iota + g * 16
        v16 = vidx[0, pl.ds(g * 16, 16)]
        vals = plsc.load_gather(buf_b, [zeros, jnp.maximum(v16 - HALF, 0)])
        plsc.store_scatter(ob, [zeros, pos], vals, mask=v16 >= HALF)
      return 0

    lax.fori_loop(0, N_GRP // 4, extract_b, 0)
    out_cps[slot] = pltpu.make_async_copy(
        ob, outt_ref.at[pl.ds(r, 1), :], osem)
    out_cps[slot].start()
    if c + 1 < ROWS_PER_W:
      @pl.when(lax.rem(r + 1, EMB_DIM) == 0)
      def _restage(nxt=(r + 1) // EMB_DIM):
        pltpu.sync_copy(catt_ref.at[pl.ds(nxt, 1), :], vidx)
      cp_bs = dma_b(r + 1)
  for cp in out_cps:
    cp.wait()

  @pl.when(wid < N_CONT)
  def _bn():
    pltpu.sync_copy(gam_ref, gb_v.at[0])
    pltpu.sync_copy(bet_ref, gb_v.at[1])
    f16 = zeros + wid
    sg = plsc.load_gather(gb_v, [zeros, f16]) * INV_STD
    sb = plsc.load_gather(gb_v, [zeros + 1, f16])
    pltpu.sync_copy(contt_ref.at[pl.ds(wid, 1), :], crow)

    def bn_group(s, _):
      pos = iota + s * 16
      v = plsc.load_gather(crow, [zeros, pos])
      plsc.store_scatter(orow_a, [zeros, pos], v * sg + sb)
      return 0

    lax.fori_loop(0, N_GRP, bn_group, 0)
    pltpu.sync_copy(orow_a, outt_ref.at[pl.ds(T_ROWS + wid, 1), :])


def kernel(cont_data, cat_data, tables, bn_gamma, bn_beta):
  t2 = tables.transpose(0, 2, 1).reshape(T_ROWS, VOCAB)
  tailp = jnp.pad(t2[:, HALF + BMAIN:], ((0, 0), (0, 128 - VTAIL)))
  catt = cat_data.astype(jnp.int32).T
  contt = cont_data.T
  gam16 = jnp.pad(bn_gamma.astype(jnp.float32), (0, 16 - N_CONT))
  bet16 = jnp.pad(bn_beta.astype(jnp.float32), (0, 16 - N_CONT))

  mesh = plsc.VectorSubcoreMesh(core_axis_name="c", subcore_axis_name="s")
  run = pl.kernel(
      _sc_body,
      out_type=jax.ShapeDtypeStruct((OUT_W, BATCH), jnp.float32),
      mesh=mesh,
      compiler_params=pltpu.CompilerParams(needs_layout_passes=False),
      scratch_types=[
          pltpu.VMEM((1, HALF), jnp.float32),
          pltpu.VMEM((1, BMAIN + 128), jnp.float32),
          pltpu.VMEM((1, BATCH), jnp.int32),
          pltpu.VMEM((1, BATCH), jnp.float32),
          pltpu.VMEM((1, BATCH), jnp.float32),
          pltpu.VMEM((1, BATCH), jnp.float32),
          pltpu.VMEM((2, 16), jnp.float32),
          pltpu.SemaphoreType.DMA,
          pltpu.SemaphoreType.DMA,
          pltpu.SemaphoreType.DMA,
      ],
  )
  outt = run(t2, tailp, catt, contt, gam16, bet16)
  return outt.T
